# baseline (speedup 1.0000x reference)
import functools

import jax
import jax.numpy as jnp
import numpy as np
from jax import lax
from jax.experimental import pallas as pl
from jax.experimental.pallas import tpu as pltpu
from jax.experimental.pallas import tpu_sc as plsc

B, C, GD, GH, GW = 8, 12, 8, 16, 16
H, W = 512, 512
NC, NS, L = 2, 16, 16
NWORK = NC * NS
ROWS = B * H
RPW = ROWS // NWORK
NG = W // L


def _x_tables():
    w = np.arange(W)
    g = (w + 0.5) * GW / W - 0.5
    f = np.floor(g).astype(np.int64)
    w1 = (g - f).astype(np.float32)
    xi0 = np.clip(f, 0, GW - 1).astype(np.int32)
    xi1 = np.clip(f + 1, 0, GW - 1).astype(np.int32)
    return xi0, xi1, w1


def _sc_body(grid_hbm, guide_hbm, xi0_hbm, xi1_hbm, wx1_hbm, out_hbm,
             grid_v, g_v, out_v, xi0_v, xi1_v, wx1_v):
    wid = lax.axis_index("s") * NC + lax.axis_index("c")
    base_row = wid * RPW
    b = base_row // H
    h0 = base_row % H

    pltpu.sync_copy(grid_hbm.at[b], grid_v)
    pltpu.sync_copy(xi0_hbm, xi0_v)
    pltpu.sync_copy(xi1_hbm, xi1_v)
    pltpu.sync_copy(wx1_hbm, wx1_v)

    def row_body(i, carry):
        h = h0 + i
        pltpu.sync_copy(guide_hbm.at[base_row + i], g_v)

        gy = (h.astype(jnp.float32) + 0.5) * (GH / H) - 0.5
        fy = (h + (H // GH) // 2) // (H // GH) - 1
        fyf = fy.astype(jnp.float32)
        wy1 = gy - fyf
        wy0 = 1.0 - wy1
        yi0 = jnp.clip(fy, 0, GH - 1)
        yi1 = jnp.clip(fy + 1, 0, GH - 1)
        ybase0 = yi0 * (GW * GD * C)
        ybase1 = yi1 * (GW * GD * C)

        def grp_body(gidx, carry2):
            w0 = gidx * L
            g16 = g_v[pl.ds(w0, L)]
            t = jnp.clip(g16 * GD - 0.5, 0.0, GD - 1.0)
            z0f = jnp.zeros((L,), jnp.float32)
            for k in range(1, GD):
                z0f = z0f + jnp.where(t >= float(k), 1.0, 0.0)
            z0 = z0f.astype(jnp.int32)
            z1 = jnp.minimum(z0 + 1, GD - 1)
            f = t - z0f
            fm = 1.0 - f

            xi0 = xi0_v[pl.ds(w0, L)]
            xi1 = xi1_v[pl.ds(w0, L)]
            wx1 = wx1_v[pl.ds(w0, L)]
            wx0 = 1.0 - wx1

            accs = [None] * C
            for (yb, wy, xi, wx) in (
                    (ybase0, wy0, xi0, wx0), (ybase0, wy0, xi1, wx1),
                    (ybase1, wy1, xi0, wx0), (ybase1, wy1, xi1, wx1)):
                wxy = wx * wy
                p = wxy * fm
                q = wxy * f
                cbase = yb + xi * (GD * C)
                i0 = cbase + z0 * C
                i1 = cbase + z1 * C
                for c in range(C):
                    v0 = plsc.load_gather(grid_v, [i0 + c])
                    v1 = plsc.load_gather(grid_v, [i1 + c])
                    term = p * v0 + q * v1
                    accs[c] = term if accs[c] is None else accs[c] + term
            for c in range(C):
                out_v[c, pl.ds(w0, L)] = accs[c]
            return carry2

        lax.fori_loop(0, NG, grp_body, 0)
        for c in range(C):
            pltpu.sync_copy(out_v.at[c], out_hbm.at[b, c, h])
        return carry

    lax.fori_loop(0, RPW, row_body, 0)


@jax.jit
def kernel(bilateral_grid, guidemap):
    grid_t = jnp.transpose(bilateral_grid, (0, 3, 4, 2, 1)).reshape(
        B, GH * GW * GD * C)
    guide = guidemap.reshape(ROWS, W)
    xi0, xi1, wx1 = _x_tables()

    mesh = plsc.VectorSubcoreMesh(core_axis_name="c", subcore_axis_name="s")
    run = functools.partial(
        pl.kernel,
        mesh=mesh,
        compiler_params=pltpu.CompilerParams(needs_layout_passes=False),
        out_type=jax.ShapeDtypeStruct((B, C, H, W), jnp.float32),
        scratch_types=[
            pltpu.VMEM((GH * GW * GD * C,), jnp.float32),
            pltpu.VMEM((W,), jnp.float32),
            pltpu.VMEM((C, W), jnp.float32),
            pltpu.VMEM((W,), jnp.int32),
            pltpu.VMEM((W,), jnp.int32),
            pltpu.VMEM((W,), jnp.float32),
        ],
    )(_sc_body)
    return run(grid_t, guide, jnp.asarray(xi0), jnp.asarray(xi1),
               jnp.asarray(wx1))


# speedup vs baseline: 2.4215x; 2.4215x over previous
import functools

import jax
import jax.numpy as jnp
import numpy as np
from jax import lax
from jax.experimental import pallas as pl
from jax.experimental.pallas import tpu as pltpu
from jax.experimental.pallas import tpu_sc as plsc

B, C, GD, GH, GW = 8, 12, 8, 16, 16
H, W = 512, 512
HT = 448
HB = 64
SB = 16
NS = HB // SB
NJ = HT // HB

NC, NSC, L = 2, 16, 16
NWORK = NC * NSC
HSC = H - HT
WPB = NWORK // B
RPW = HSC // WPB
NG = W // L


def _interp_matrix(npix, ncell):
    w = np.arange(npix)
    g = (w + 0.5) * ncell / npix - 0.5
    f = np.floor(g).astype(np.int64)
    w1 = (g - f).astype(np.float32)
    w0 = 1.0 - w1
    m = np.zeros((ncell, npix), np.float32)
    np.add.at(m, (np.clip(f, 0, ncell - 1), w), w0)
    np.add.at(m, (np.clip(f + 1, 0, ncell - 1), w), w1)
    return m


def _x_tables():
    w = np.arange(W)
    g = (w + 0.5) * GW / W - 0.5
    f = np.floor(g).astype(np.int64)
    w1 = (g - f).astype(np.float32)
    xi0 = np.clip(f, 0, GW - 1).astype(np.int32)
    xi1 = np.clip(f + 1, 0, GW - 1).astype(np.int32)
    return xi0, xi1, w1


def _tc_body(g5_ref, rxt_ref, guide_ref, out_ref, u_ref):
    j = pl.program_id(1)

    @pl.when(j == 0)
    def _():
        u = jnp.dot(g5_ref[0], rxt_ref[...],
                    preferred_element_type=jnp.float32)
        u_ref[...] = u.reshape(GH, C, GD, W)

    for s in range(NS):
        band = j * NS + s
        fy = (band - 1) // 2
        yi0 = jnp.clip(fy, 0, GH - 1)
        yi1 = jnp.clip(fy + 1, 0, GH - 1)
        hrow = (jax.lax.broadcasted_iota(jnp.int32, (SB, 1), 0)
                + band * SB).astype(jnp.float32)
        gy = (hrow + 0.5) * (GH / H) - 0.5
        wy1 = gy - fy.astype(jnp.float32)
        wy0 = 1.0 - wy1

        u0 = u_ref[yi0]
        u1 = u_ref[yi1]

        g = guide_ref[0, s * SB:(s + 1) * SB]
        t = jnp.clip(g * GD - 0.5, 0.0, GD - 1.0)
        wz = [jnp.maximum(1.0 - jnp.abs(t - k), 0.0) for k in range(GD)]

        for c in range(C):
            a0 = wz[0] * u0[c, 0][None, :]
            a1 = wz[0] * u1[c, 0][None, :]
            for k in range(1, GD):
                a0 = a0 + wz[k] * u0[c, k][None, :]
                a1 = a1 + wz[k] * u1[c, k][None, :]
            out_ref[0, c, s * SB:(s + 1) * SB] = wy0 * a0 + wy1 * a1


def _tc_call(g5, rxt, guide):
    return pl.pallas_call(
        _tc_body,
        grid=(B, NJ),
        in_specs=[
            pl.BlockSpec((1, GH * C * GD, GW), lambda b, j: (b, 0, 0)),
            pl.BlockSpec((GW, W), lambda b, j: (0, 0)),
            pl.BlockSpec((1, HB, W), lambda b, j: (b, j, 0)),
        ],
        out_specs=pl.BlockSpec((1, C, HB, W), lambda b, j: (b, 0, j, 0)),
        out_shape=jax.ShapeDtypeStruct((B, C, HT, W), jnp.float32),
        scratch_shapes=[pltpu.VMEM((GH, C, GD, W), jnp.float32)],
    )(g5, rxt, guide)


def _sc_body(grid_hbm, guide_hbm, xi0_hbm, xi1_hbm, wx1_hbm, out_hbm,
             grid_v, g_v, out_v, xi0_v, xi1_v, wx1_v):
    wid = lax.axis_index("s") * NC + lax.axis_index("c")
    b = wid // WPB
    h0 = HT + (wid % WPB) * RPW

    pltpu.sync_copy(grid_hbm.at[b], grid_v)
    pltpu.sync_copy(xi0_hbm, xi0_v)
    pltpu.sync_copy(xi1_hbm, xi1_v)
    pltpu.sync_copy(wx1_hbm, wx1_v)

    def row_body(i, carry):
        h = h0 + i
        pltpu.sync_copy(guide_hbm.at[b, h], g_v)

        gy = (h.astype(jnp.float32) + 0.5) * (GH / H) - 0.5
        fy = (h + (H // GH) // 2) // (H // GH) - 1
        fyf = fy.astype(jnp.float32)
        wy1 = gy - fyf
        wy0 = 1.0 - wy1
        yi0 = jnp.clip(fy, 0, GH - 1)
        yi1 = jnp.clip(fy + 1, 0, GH - 1)
        ybase0 = yi0 * (GW * GD * C)
        ybase1 = yi1 * (GW * GD * C)

        def grp_body(gidx, carry2):
            w0 = gidx * L
            g16 = g_v[pl.ds(w0, L)]
            t = jnp.clip(g16 * GD - 0.5, 0.0, GD - 1.0)
            z0f = jnp.zeros((L,), jnp.float32)
            for k in range(1, GD):
                z0f = z0f + jnp.where(t >= float(k), 1.0, 0.0)
            z0 = z0f.astype(jnp.int32)
            z1 = jnp.minimum(z0 + 1, GD - 1)
            f = t - z0f
            fm = 1.0 - f

            xi0 = xi0_v[pl.ds(w0, L)]
            xi1 = xi1_v[pl.ds(w0, L)]
            wx1 = wx1_v[pl.ds(w0, L)]
            wx0 = 1.0 - wx1

            accs = [None] * C
            for (yb, wy, xi, wx) in (
                    (ybase0, wy0, xi0, wx0), (ybase0, wy0, xi1, wx1),
                    (ybase1, wy1, xi0, wx0), (ybase1, wy1, xi1, wx1)):
                wxy = wx * wy
                p = wxy * fm
                q = wxy * f
                cbase = yb + xi * (GD * C)
                i0 = cbase + z0 * C
                i1 = cbase + z1 * C
                for c in range(C):
                    v0 = plsc.load_gather(grid_v, [i0 + c])
                    v1 = plsc.load_gather(grid_v, [i1 + c])
                    term = p * v0 + q * v1
                    accs[c] = term if accs[c] is None else accs[c] + term
            for c in range(C):
                out_v[c, pl.ds(w0, L)] = accs[c]
            return carry2

        lax.fori_loop(0, NG, grp_body, 0)
        for c in range(C):
            pltpu.sync_copy(out_v.at[c], out_hbm.at[b, c, h - HT])
        return carry

    lax.fori_loop(0, RPW, row_body, 0)


def _sc_call(grid_t, guide, xi0, xi1, wx1):
    mesh = plsc.VectorSubcoreMesh(core_axis_name="c", subcore_axis_name="s")
    run = functools.partial(
        pl.kernel,
        mesh=mesh,
        compiler_params=pltpu.CompilerParams(needs_layout_passes=False),
        out_type=jax.ShapeDtypeStruct((B, C, HSC, W), jnp.float32),
        scratch_types=[
            pltpu.VMEM((GH * GW * GD * C,), jnp.float32),
            pltpu.VMEM((W,), jnp.float32),
            pltpu.VMEM((C, W), jnp.float32),
            pltpu.VMEM((W,), jnp.int32),
            pltpu.VMEM((W,), jnp.int32),
            pltpu.VMEM((W,), jnp.float32),
        ],
    )(_sc_body)
    return run(grid_t, guide, xi0, xi1, wx1)


@jax.jit
def kernel(bilateral_grid, guidemap):
    guide = guidemap.reshape(B, H, W)

    g5 = jnp.transpose(bilateral_grid, (0, 3, 1, 2, 4)).reshape(
        B, GH * C * GD, GW)
    rxt = jnp.asarray(_interp_matrix(W, GW))
    out_tc = _tc_call(g5, rxt, guide[:, :HT])

    grid_t = jnp.transpose(bilateral_grid, (0, 3, 4, 2, 1)).reshape(
        B, GH * GW * GD * C)
    xi0, xi1, wx1 = _x_tables()
    out_sc = _sc_call(grid_t, guide, jnp.asarray(xi0), jnp.asarray(xi1),
                      jnp.asarray(wx1))

    return jnp.concatenate([out_tc, out_sc], axis=2)


# speedup vs baseline: 4.0662x; 1.6792x over previous
import jax
import jax.numpy as jnp
import numpy as np
from jax.experimental import pallas as pl
from jax.experimental.pallas import tpu as pltpu

B, C, GD, GH, GW = 8, 12, 8, 16, 16
H, W = 512, 512
HB = 256
SB = 16
NS = HB // SB
NJ = H // HB


def _interp_matrix(npix, ncell):
    w = np.arange(npix)
    g = (w + 0.5) * ncell / npix - 0.5
    f = np.floor(g).astype(np.int64)
    w1 = (g - f).astype(np.float32)
    w0 = 1.0 - w1
    m = np.zeros((ncell, npix), np.float32)
    np.add.at(m, (np.clip(f, 0, ncell - 1), w), w0)
    np.add.at(m, (np.clip(f + 1, 0, ncell - 1), w), w1)
    return m


def _body(g5_ref, rxt_ref, guide_ref, out_ref, u_ref):
    j = pl.program_id(1)

    @pl.when(j == 0)
    def _():
        u = jnp.dot(g5_ref[0], rxt_ref[...],
                    preferred_element_type=jnp.float32)
        u_ref[...] = u.reshape(GH, C, GD, W)

    for s in range(NS):
        band = j * NS + s
        fy = (band - 1) // 2
        yi0 = jnp.clip(fy, 0, GH - 1)
        yi1 = jnp.clip(fy + 1, 0, GH - 1)
        hrow = (jax.lax.broadcasted_iota(jnp.int32, (SB, 1), 0)
                + band * SB).astype(jnp.float32)
        gy = (hrow + 0.5) * (GH / H) - 0.5
        wy1 = gy - fy.astype(jnp.float32)
        wy0 = 1.0 - wy1

        u0 = u_ref[yi0]
        u1 = u_ref[yi1]

        g = guide_ref[0, s * SB:(s + 1) * SB]
        t = jnp.clip(g * GD - 0.5, 0.0, GD - 1.0)
        wz = [jnp.maximum(1.0 - jnp.abs(t - k), 0.0) for k in range(GD)]

        for c in range(C):
            a0 = wz[0] * u0[c, 0][None, :]
            a1 = wz[0] * u1[c, 0][None, :]
            for k in range(1, GD):
                a0 = a0 + wz[k] * u0[c, k][None, :]
                a1 = a1 + wz[k] * u1[c, k][None, :]
            out_ref[0, c, s * SB:(s + 1) * SB] = wy0 * a0 + wy1 * a1


@jax.jit
def kernel(bilateral_grid, guidemap):
    g5 = jnp.transpose(bilateral_grid, (0, 3, 1, 2, 4)).reshape(B, GH * C * GD, GW)
    rxt = jnp.asarray(_interp_matrix(W, GW))
    guide = guidemap.reshape(B, H, W)

    return pl.pallas_call(
        _body,
        grid=(B, NJ),
        in_specs=[
            pl.BlockSpec((1, GH * C * GD, GW), lambda b, j: (b, 0, 0)),
            pl.BlockSpec((GW, W), lambda b, j: (0, 0)),
            pl.BlockSpec((1, HB, W), lambda b, j: (b, j, 0)),
        ],
        out_specs=pl.BlockSpec((1, C, HB, W), lambda b, j: (b, 0, j, 0)),
        out_shape=jax.ShapeDtypeStruct((B, C, H, W), jnp.float32),
        scratch_shapes=[pltpu.VMEM((GH, C, GD, W), jnp.float32)],
    )(g5, rxt, guide)


# speedup vs baseline: 4.0856x; 1.0048x over previous
import jax
import jax.numpy as jnp
import numpy as np
from jax.experimental import pallas as pl
from jax.experimental.pallas import tpu as pltpu

B, C, GD, GH, GW = 8, 12, 8, 16, 16
H, W = 512, 512
HB = 128
SB = 16
NS = HB // SB
NJ = H // HB


def _interp_matrix(npix, ncell):
    w = np.arange(npix)
    g = (w + 0.5) * ncell / npix - 0.5
    f = np.floor(g).astype(np.int64)
    w1 = (g - f).astype(np.float32)
    w0 = 1.0 - w1
    m = np.zeros((ncell, npix), np.float32)
    np.add.at(m, (np.clip(f, 0, ncell - 1), w), w0)
    np.add.at(m, (np.clip(f + 1, 0, ncell - 1), w), w1)
    return m


def _body(g5_ref, rxt_ref, guide_ref, out_ref, u_ref):
    j = pl.program_id(1)

    @pl.when(j == 0)
    def _():
        u = jnp.dot(g5_ref[0], rxt_ref[...],
                    preferred_element_type=jnp.float32)
        u_ref[...] = u.reshape(GH, C, GD, W)

    for s in range(NS):
        band = j * NS + s
        fy = (band - 1) // 2
        yi0 = jnp.clip(fy, 0, GH - 1)
        yi1 = jnp.clip(fy + 1, 0, GH - 1)
        hrow = (jax.lax.broadcasted_iota(jnp.int32, (SB, 1), 0)
                + band * SB).astype(jnp.float32)
        gy = (hrow + 0.5) * (GH / H) - 0.5
        wy1 = gy - fy.astype(jnp.float32)
        wy0 = 1.0 - wy1

        u0 = u_ref[yi0]
        u1 = u_ref[yi1]

        g = guide_ref[0, s * SB:(s + 1) * SB]
        t = jnp.clip(g * GD - 0.5, 0.0, GD - 1.0)
        wz = [jnp.maximum(1.0 - jnp.abs(t - k), 0.0) for k in range(GD)]

        for c in range(C):
            a0 = wz[0] * u0[c, 0][None, :]
            a1 = wz[0] * u1[c, 0][None, :]
            for k in range(1, GD):
                a0 = a0 + wz[k] * u0[c, k][None, :]
                a1 = a1 + wz[k] * u1[c, k][None, :]
            out_ref[0, c, s * SB:(s + 1) * SB] = wy0 * a0 + wy1 * a1


@jax.jit
def kernel(bilateral_grid, guidemap):
    g5 = jnp.transpose(bilateral_grid, (0, 3, 1, 2, 4)).reshape(B, GH * C * GD, GW)
    rxt = jnp.asarray(_interp_matrix(W, GW))
    guide = guidemap.reshape(B, H, W)

    return pl.pallas_call(
        _body,
        grid=(B, NJ),
        in_specs=[
            pl.BlockSpec((1, GH * C * GD, GW), lambda b, j: (b, 0, 0)),
            pl.BlockSpec((GW, W), lambda b, j: (0, 0)),
            pl.BlockSpec((1, HB, W), lambda b, j: (b, j, 0)),
        ],
        out_specs=pl.BlockSpec((1, C, HB, W), lambda b, j: (b, 0, j, 0)),
        out_shape=jax.ShapeDtypeStruct((B, C, H, W), jnp.float32),
        scratch_shapes=[pltpu.VMEM((GH, C, GD, W), jnp.float32)],
    )(g5, rxt, guide)


# speedup vs baseline: 4.6870x; 1.1472x over previous
import jax
import jax.numpy as jnp
import numpy as np
from jax.experimental import pallas as pl
from jax.experimental.pallas import tpu as pltpu

B, C, GD, GH, GW = 8, 12, 8, 16, 16
H, W = 512, 512
HB = 128
SB = 16
NS = HB // SB
NJ = H // HB


def _interp_matrix(npix, ncell):
    w = np.arange(npix)
    g = (w + 0.5) * ncell / npix - 0.5
    f = np.floor(g).astype(np.int64)
    w1 = (g - f).astype(np.float32)
    w0 = 1.0 - w1
    m = np.zeros((ncell, npix), np.float32)
    np.add.at(m, (np.clip(f, 0, ncell - 1), w), w0)
    np.add.at(m, (np.clip(f + 1, 0, ncell - 1), w), w1)
    return m


def _body(g5_ref, rxt_ref, guide_ref, out_ref, u_ref):
    j = pl.program_id(1)

    @pl.when(j == 0)
    def _():
        u = jnp.dot(g5_ref[0], rxt_ref[...],
                    preferred_element_type=jnp.float32)
        ub = u.astype(jnp.bfloat16).reshape(GH, C, GD, 1, W)
        u_ref[...] = jnp.broadcast_to(ub, (GH, C, GD, SB, W))

    for s in range(NS):
        band = j * NS + s
        fy = (band - 1) // 2
        yi0 = jnp.clip(fy, 0, GH - 1)
        yi1 = jnp.clip(fy + 1, 0, GH - 1)
        hrow = (jax.lax.broadcasted_iota(jnp.int32, (SB, 1), 0)
                + band * SB).astype(jnp.float32)
        gy = (hrow + 0.5) * (GH / H) - 0.5
        wy1 = gy - fy.astype(jnp.float32)
        wy0 = 1.0 - wy1

        u0 = u_ref[yi0]
        u1 = u_ref[yi1]

        g = guide_ref[0, s * SB:(s + 1) * SB]
        t = jnp.clip(g * GD - 0.5, 0.0, GD - 1.0)
        wz = [jnp.maximum(1.0 - jnp.abs(t - k), 0.0).astype(jnp.bfloat16)
              for k in range(GD)]

        for c in range(C):
            a0 = wz[0] * u0[c, 0]
            a1 = wz[0] * u1[c, 0]
            for k in range(1, GD):
                a0 = a0 + wz[k] * u0[c, k]
                a1 = a1 + wz[k] * u1[c, k]
            out_ref[0, c, s * SB:(s + 1) * SB] = (
                wy0 * a0.astype(jnp.float32) + wy1 * a1.astype(jnp.float32))


@jax.jit
def kernel(bilateral_grid, guidemap):
    g5 = jnp.transpose(bilateral_grid, (0, 3, 1, 2, 4)).reshape(B, GH * C * GD, GW)
    rxt = jnp.asarray(_interp_matrix(W, GW))
    guide = guidemap.reshape(B, H, W)

    return pl.pallas_call(
        _body,
        grid=(B, NJ),
        in_specs=[
            pl.BlockSpec((1, GH * C * GD, GW), lambda b, j: (b, 0, 0)),
            pl.BlockSpec((GW, W), lambda b, j: (0, 0)),
            pl.BlockSpec((1, HB, W), lambda b, j: (b, j, 0)),
        ],
        out_specs=pl.BlockSpec((1, C, HB, W), lambda b, j: (b, 0, j, 0)),
        out_shape=jax.ShapeDtypeStruct((B, C, H, W), jnp.float32),
        scratch_shapes=[pltpu.VMEM((GH, C, GD, SB, W), jnp.bfloat16)],
    )(g5, rxt, guide)


# speedup vs baseline: 4.9958x; 1.0659x over previous
import jax
import jax.numpy as jnp
import numpy as np
from jax.experimental import pallas as pl
from jax.experimental.pallas import tpu as pltpu

B, C, GD, GH, GW = 8, 12, 8, 16, 16
H, W = 512, 512
HB = 128
SB = 16
NS = HB // SB
NJ = H // HB


def _interp_matrix(npix, ncell):
    w = np.arange(npix)
    g = (w + 0.5) * ncell / npix - 0.5
    f = np.floor(g).astype(np.int64)
    w1 = (g - f).astype(np.float32)
    w0 = 1.0 - w1
    m = np.zeros((ncell, npix), np.float32)
    np.add.at(m, (np.clip(f, 0, ncell - 1), w), w0)
    np.add.at(m, (np.clip(f + 1, 0, ncell - 1), w), w1)
    return m


def _body(g5_ref, rxt_ref, guide_ref, out_ref, u_ref):
    j = pl.program_id(1)

    @pl.when(j == 0)
    def _():
        u = jnp.dot(g5_ref[0], rxt_ref[...],
                    preferred_element_type=jnp.float32)
        ub = u.astype(jnp.bfloat16).reshape(GH, C, GD, 1, W)
        u_ref[...] = jnp.broadcast_to(ub, (GH, C, GD, SB, W))

    for s in range(NS):
        band = j * NS + s
        fy = (band - 1) // 2
        yi0 = jnp.clip(fy, 0, GH - 1)
        yi1 = jnp.clip(fy + 1, 0, GH - 1)
        hrow = (jax.lax.broadcasted_iota(jnp.int32, (SB, 1), 0)
                + band * SB).astype(jnp.float32)
        gy = (hrow + 0.5) * (GH / H) - 0.5
        wy1 = (gy - fy.astype(jnp.float32)).astype(jnp.bfloat16)
        wy0 = (1.0 - wy1.astype(jnp.float32)).astype(jnp.bfloat16)

        u0 = u_ref[yi0]
        u1 = u_ref[yi1]

        g = guide_ref[0, s * SB:(s + 1) * SB]
        t = jnp.clip(g * GD - 0.5, 0.0, GD - 1.0)
        wz = [jnp.maximum(1.0 - jnp.abs(t - k), 0.0).astype(jnp.bfloat16)
              for k in range(GD)]

        for c in range(C):
            a0 = wz[0] * u0[c, 0]
            a1 = wz[0] * u1[c, 0]
            for k in range(1, GD):
                a0 = a0 + wz[k] * u0[c, k]
                a1 = a1 + wz[k] * u1[c, k]
            out_ref[0, c, s * SB:(s + 1) * SB] = (
                wy0 * a0 + wy1 * a1).astype(jnp.float32)


@jax.jit
def kernel(bilateral_grid, guidemap):
    g5 = jnp.transpose(bilateral_grid, (0, 3, 1, 2, 4)).reshape(B, GH * C * GD, GW)
    rxt = jnp.asarray(_interp_matrix(W, GW))
    guide = guidemap.reshape(B, H, W)

    return pl.pallas_call(
        _body,
        grid=(B, NJ),
        in_specs=[
            pl.BlockSpec((1, GH * C * GD, GW), lambda b, j: (b, 0, 0)),
            pl.BlockSpec((GW, W), lambda b, j: (0, 0)),
            pl.BlockSpec((1, HB, W), lambda b, j: (b, j, 0)),
        ],
        out_specs=pl.BlockSpec((1, C, HB, W), lambda b, j: (b, 0, j, 0)),
        out_shape=jax.ShapeDtypeStruct((B, C, H, W), jnp.float32),
        scratch_shapes=[pltpu.VMEM((GH, C, GD, SB, W), jnp.bfloat16)],
    )(g5, rxt, guide)
